# baseline (speedup 1.0000x reference)
import functools

import numpy as np
import jax
import jax.numpy as jnp
from jax import lax
from jax.experimental import pallas as pl
from jax.experimental.pallas import tpu as pltpu
from jax.experimental.pallas import tpu_sc as plsc

_N = 10000
_E = 160000
_C = 128
_L2 = 4
_NELEM = 4
_ROW = _L2 * _C
_AVG = 16.0

_NC, _NS = 2, 16
_NW = _NC * _NS
_EPT = _E // _NW
_B = 32
_NB = (_EPT + _B - 1) // _B
_NP = 5
_RP = 2048
_NPAD = _NP * _RP
_RPS = _RP // _NS
_EPAD = _E + 2 * _B
_ZROW = _N
_SENT = 1 << 30


def _sc_messages(h_pad, ea_exp, ef, rcv_a, gid_a, snd_a, rc0, rc1, rc2,
                 rc3):
    mesh = plsc.VectorSubcoreMesh(core_axis_name="c", subcore_axis_name="s")
    scratch = [
        pltpu.VMEM((_EPT + 32,), jnp.int32),
        pltpu.VMEM((_B,), jnp.int32),
        pltpu.VMEM((_B,), jnp.int32),
        pltpu.VMEM((_B,), jnp.int32),
        pltpu.VMEM((_B,), jnp.int32),
        pltpu.VMEM((_B,), jnp.int32),
        pltpu.VMEM((_B,), jnp.int32),
        pltpu.VMEM((_B, _ROW), jnp.float32),
        pltpu.VMEM((_B, _C), jnp.float32),
        pltpu.VMEM((_B, _C), jnp.float32),
        pltpu.VMEM((_B, _C), jnp.float32),
        pltpu.VMEM((_B, _C), jnp.float32),
        pltpu.VMEM((_B, _C), jnp.float32),
        pltpu.VMEM((_B, _C), jnp.float32),
        pltpu.VMEM_SHARED((_RP * _L2, _C), jnp.float32),
        pltpu.SemaphoreType.DMA,
        pltpu.SemaphoreType.DMA,
        pltpu.SemaphoreType.DMA,
    ]

    @functools.partial(
        pl.kernel,
        out_type=jax.ShapeDtypeStruct((_NC, _NPAD * _L2, _C), jnp.float32),
        mesh=mesh,
        scratch_types=scratch,
    )
    def k(h_hbm, ea_hbm, ef_hbm, rcv_hbm, gid_hbm, snd_hbm,
          rc0_hbm, rc1_hbm, rc2_hbm, rc3_hbm, z_hbm, out_hbm,
          recv_s, gid_b, snd_b, rc0_b, rc1_b, rc2_b, rc3_b,
          ef_b, h_b, ea_b, m0_b, m1_b, m2_b, m3_b,
          msg_sh, sem0, sem1, sem2):
        cid = lax.axis_index("c")
        sid = lax.axis_index("s")
        wid = cid * _NS + sid
        ebase = wid * _EPT
        iota16 = lax.iota(jnp.int32, 16)

        sent16 = jnp.full((16,), _SENT, jnp.int32)
        recv_s[pl.ds(_EPT - 8, 16)] = sent16
        recv_s[pl.ds(_EPT + 8, 16)] = sent16
        pltpu.sync_copy(rcv_hbm.at[pl.ds(ebase, _EPT)],
                        recv_s.at[pl.ds(0, _EPT)])

        for p in range(_NP):
            lo = p * _RP
            hi = lo + _RP

            plsc.subcore_barrier()
            pltpu.sync_copy(z_hbm.at[pl.ds(sid * (_RPS * _L2), _RPS * _L2)],
                            msg_sh.at[pl.ds(sid * (_RPS * _L2),
                                            _RPS * _L2)])
            plsc.subcore_barrier()

            lov = jnp.full((16,), lo, jnp.int32)
            hiv = jnp.full((16,), hi, jnp.int32)

            def scan_lo(i, below):
                rv = recv_s[pl.ds(i * 16, 16)]
                msk = rv < lov
                return below + jnp.sum(msk.astype(jnp.int32))

            def scan_act(i, act):
                rv = recv_s[pl.ds(i * 16, 16)]
                msk = (rv >= lov) & (rv < hiv)
                return act + jnp.sum(msk.astype(jnp.int32))

            nstart = jnp.int32(0)
            nend = jnp.int32(_NB)

            def batch_body(j, carry):
                base = j * _B
                if True:
                    off = pl.multiple_of(ebase + base, 8)
                    c0 = pltpu.async_copy(gid_hbm.at[pl.ds(off, _B)],
                                          gid_b, sem0)
                    c1 = pltpu.async_copy(snd_hbm.at[pl.ds(off, _B)],
                                          snd_b, sem1)
                    c2 = pltpu.async_copy(rc0_hbm.at[pl.ds(off, _B)],
                                          rc0_b, sem2)
                    c3 = pltpu.async_copy(rc1_hbm.at[pl.ds(off, _B)],
                                          rc1_b, sem0)
                    c4 = pltpu.async_copy(rc2_hbm.at[pl.ds(off, _B)],
                                          rc2_b, sem1)
                    c5 = pltpu.async_copy(rc3_hbm.at[pl.ds(off, _B)],
                                          rc3_b, sem2)
                    c0.wait()
                    c1.wait()
                    c2.wait()
                    c3.wait()
                    c4.wait()
                    c5.wait()
                    lov2 = jnp.full((16,), lo, jnp.int32)
                    hiv2 = jnp.full((16,), hi, jnp.int32)
                    zrow2 = jnp.full((16,), _ZROW, jnp.int32)
                    for kk in range(_B // 16):
                        rv = recv_s[pl.ds(base + kk * 16, 16)]
                        m = (rv >= lov2) & (rv < hiv2)
                        sv = snd_b[pl.ds(kk * 16, 16)]
                        snd_b[pl.ds(kk * 16, 16)] = jnp.where(m, sv, zrow2)
                    g0 = pltpu.async_copy(ef_hbm.at[gid_b], ef_b, sem0)
                    g1 = pltpu.async_copy(h_hbm.at[snd_b], h_b, sem1)
                    g2 = pltpu.async_copy(ea_hbm.at[gid_b], ea_b, sem2)
                    g0.wait()
                    g1.wait()
                    g2.wait()

                    m_bufs = [m0_b, m1_b, m2_b, m3_b]

                    def edge_body(e, carry2):
                        hv = [h_b[e, pl.ds(c * 16, 16)]
                              for c in range(_C // 16)]
                        for l in range(_L2):
                            eav = ea_b[e, pl.ds(l * 16, 16)]
                            for c in range(_C // 16):
                                o = l * _C + c * 16
                                m_bufs[l][e, pl.ds(c * 16, 16)] = (
                                    ef_b[e, pl.ds(o, 16)] * hv[c] * eav)
                        return carry2

                    lax.fori_loop(0, _B, edge_body, 0)
                    pltpu.sync_copy(m0_b, msg_sh.at[rc0_b], add=True)
                    pltpu.sync_copy(m1_b, msg_sh.at[rc1_b], add=True)
                    pltpu.sync_copy(m2_b, msg_sh.at[rc2_b], add=True)
                    pltpu.sync_copy(m3_b, msg_sh.at[rc3_b], add=True)

                return carry

            lax.fori_loop(nstart, nend, batch_body, 0)

            plsc.subcore_barrier()
            r0 = pl.multiple_of(sid * (_RPS * _L2), 8)
            pltpu.sync_copy(
                msg_sh.at[pl.ds(r0, _RPS * _L2)],
                out_hbm.at[cid, pl.ds(p * (_RP * _L2) + r0, _RPS * _L2)])

    zeros_hbm = jnp.zeros((_RP * _L2, _C), jnp.float32)
    return k(h_pad, ea_exp, ef, rcv_a, gid_a, snd_a, rc0, rc1, rc2, rc3,
             zeros_hbm)


def _matmul_h(node_feats, W_up):
    rb = 1000

    def body(x_ref, w_ref, o_ref):
        o_ref[...] = jnp.dot(x_ref[...], w_ref[...],
                             preferred_element_type=jnp.float32)

    return pl.pallas_call(
        body,
        grid=(_N // rb,),
        in_specs=[pl.BlockSpec((rb, _C), lambda i: (i, 0)),
                  pl.BlockSpec((_C, _C), lambda i: (0, 0))],
        out_specs=pl.BlockSpec((rb, _C), lambda i: (i, 0)),
        out_shape=jax.ShapeDtypeStruct((_N, _C), jnp.float32),
    )(node_feats, W_up)


def _combine_weights(W_lin, W_skip):
    scale = 1.0 / (np.sqrt(_C) * _AVG * np.sqrt(_NELEM * _C))
    lof = [0, 1, 1, 1]

    def body(wl_ref, ws_ref, o_ref):
        for l in range(_L2):
            lw = wl_ref[lof[l]] * scale
            for e in range(_NELEM):
                o_ref[l, :, e * _C:(e + 1) * _C] = jnp.dot(
                    lw, ws_ref[e, lof[l]],
                    preferred_element_type=jnp.float32)

    return pl.pallas_call(
        body,
        out_shape=jax.ShapeDtypeStruct((_L2, _C, _NELEM * _C), jnp.float32),
    )(W_lin, W_skip)


def _final(partials, node_attrs, Wfull):
    rb = 1000

    def body(p_ref, a_ref, w_ref, o_ref):
        msg = p_ref[0] + p_ref[1]
        a = a_ref[...]
        for l in range(_L2):
            msg_l = msg[:, l * _C:(l + 1) * _C]
            y = jnp.dot(msg_l, w_ref[l], preferred_element_type=jnp.float32)
            acc = a[:, 0:1] * y[:, :_C]
            for e in range(1, _NELEM):
                acc = acc + a[:, e:e + 1] * y[:, e * _C:(e + 1) * _C]
            o_ref[:, l, :] = acc

    return pl.pallas_call(
        body,
        grid=(_N // rb,),
        in_specs=[pl.BlockSpec((_NC, rb, _ROW), lambda i: (0, i, 0)),
                  pl.BlockSpec((rb, _NELEM), lambda i: (i, 0)),
                  pl.BlockSpec((_L2, _C, _NELEM * _C), lambda i: (0, 0, 0))],
        out_specs=pl.BlockSpec((rb, _L2, _C), lambda i: (i, 0, 0)),
        out_shape=jax.ShapeDtypeStruct((_N, _L2, _C), jnp.float32),
    )(partials, node_attrs, Wfull)


def kernel(node_attrs, node_feats, edge_attrs, edge_feats, edge_index,
           W_up, W_lin, W_skip):
    h = _matmul_h(node_feats, W_up)
    h_pad = jnp.pad(h, ((0, 8), (0, 0)))
    ea_exp = jnp.pad(jnp.repeat(edge_attrs, 16, axis=1),
                     ((0, 0), (0, _C - 16 * _L2)))
    recv = edge_index[0]
    perm = jnp.argsort(recv).astype(jnp.int32)
    rcv_s = jnp.take(recv, perm)
    snd_s = jnp.take(edge_index[1], perm)
    rloc4 = (rcv_s % _RP) * _L2

    def pad_e(a, v):
        return jnp.pad(a, (0, _EPAD - _E), constant_values=v)

    gid_a = pad_e(perm, 0)
    snd_a = pad_e(snd_s, _ZROW)
    rcs = [pad_e(rloc4 + l, l) for l in range(_L2)]
    flat = _sc_messages(h_pad, ea_exp, edge_feats, rcv_s, gid_a, snd_a,
                        rcs[0], rcs[1], rcs[2], rcs[3])
    partials = flat.reshape(_NC, _NPAD, _ROW)
    Wfull = _combine_weights(W_lin, W_skip)
    return _final(partials, node_attrs, Wfull)


# speedup vs baseline: 1.0001x; 1.0001x over previous
import functools

import numpy as np
import jax
import jax.numpy as jnp
from jax import lax
from jax.experimental import pallas as pl
from jax.experimental.pallas import tpu as pltpu
from jax.experimental.pallas import tpu_sc as plsc

_N = 10000
_E = 160000
_C = 128
_L2 = 4
_NELEM = 4
_ROW = _L2 * _C
_AVG = 16.0

_NC, _NS = 2, 16
_NW = _NC * _NS
_EPT = _E // _NW
_B = 32
_NB = (_EPT + _B - 1) // _B
_NP = 5
_RP = 2048
_NPAD = _NP * _RP
_RPS = _RP // _NS
_EPAD = _E + 2 * _B
_ZROW = _N
_SENT = 1 << 30


def _sc_messages(h_pad, ea_exp, ef, rcv_a, gid_a, snd_a, rc0, rc1, rc2,
                 rc3):
    mesh = plsc.VectorSubcoreMesh(core_axis_name="c", subcore_axis_name="s")
    scratch = [
        pltpu.VMEM((_EPT + 32,), jnp.int32),
        pltpu.VMEM((_B,), jnp.int32),
        pltpu.VMEM((_B,), jnp.int32),
        pltpu.VMEM((_B,), jnp.int32),
        pltpu.VMEM((_B,), jnp.int32),
        pltpu.VMEM((_B,), jnp.int32),
        pltpu.VMEM((_B,), jnp.int32),
        pltpu.VMEM((_B, _ROW), jnp.float32),
        pltpu.VMEM((_B, _C), jnp.float32),
        pltpu.VMEM((_B, _C), jnp.float32),
        pltpu.VMEM((_B, _C), jnp.float32),
        pltpu.VMEM((_B, _C), jnp.float32),
        pltpu.VMEM((_B, _C), jnp.float32),
        pltpu.VMEM((_B, _C), jnp.float32),
        pltpu.VMEM_SHARED((_RP * _L2, _C), jnp.float32),
        pltpu.SemaphoreType.DMA,
        pltpu.SemaphoreType.DMA,
        pltpu.SemaphoreType.DMA,
        pltpu.SemaphoreType.DMA,
    ]

    @functools.partial(
        pl.kernel,
        out_type=jax.ShapeDtypeStruct((_NC, _NPAD * _L2, _C), jnp.float32),
        mesh=mesh,
        scratch_types=scratch,
    )
    def k(h_hbm, ea_hbm, ef_hbm, rcv_hbm, gid_hbm, snd_hbm,
          rc0_hbm, rc1_hbm, rc2_hbm, rc3_hbm, z_hbm, out_hbm,
          recv_s, gid_b, snd_b, rc0_b, rc1_b, rc2_b, rc3_b,
          ef_b, h_b, ea_b, m0_b, m1_b, m2_b, m3_b,
          msg_sh, sem0, sem1, sem2, sem3):
        cid = lax.axis_index("c")
        sid = lax.axis_index("s")
        wid = cid * _NS + sid
        ebase = wid * _EPT
        iota16 = lax.iota(jnp.int32, 16)

        sent16 = jnp.full((16,), _SENT, jnp.int32)
        recv_s[pl.ds(_EPT - 8, 16)] = sent16
        recv_s[pl.ds(_EPT + 8, 16)] = sent16
        pltpu.sync_copy(rcv_hbm.at[pl.ds(ebase, _EPT)],
                        recv_s.at[pl.ds(0, _EPT)])

        for p in range(_NP):
            lo = p * _RP
            hi = lo + _RP

            plsc.subcore_barrier()
            pltpu.sync_copy(z_hbm.at[pl.ds(sid * (_RPS * _L2), _RPS * _L2)],
                            msg_sh.at[pl.ds(sid * (_RPS * _L2),
                                            _RPS * _L2)])
            plsc.subcore_barrier()

            lov = jnp.full((16,), lo, jnp.int32)
            hiv = jnp.full((16,), hi, jnp.int32)

            def scan_lo(i, below):
                rv = recv_s[pl.ds(i * 16, 16)]
                msk = rv < lov
                return below + jnp.sum(msk.astype(jnp.int32))

            def scan_act(i, act):
                rv = recv_s[pl.ds(i * 16, 16)]
                msk = (rv >= lov) & (rv < hiv)
                return act + jnp.sum(msk.astype(jnp.int32))

            nstart = jnp.int32(0)
            nend = jnp.int32(_NB)

            def batch_body(j, carry):
                base = j * _B
                if True:
                    off = pl.multiple_of(ebase + base, 8)
                    c0 = pltpu.async_copy(gid_hbm.at[pl.ds(off, _B)],
                                          gid_b, sem0)
                    c1 = pltpu.async_copy(snd_hbm.at[pl.ds(off, _B)],
                                          snd_b, sem1)
                    c2 = pltpu.async_copy(rc0_hbm.at[pl.ds(off, _B)],
                                          rc0_b, sem2)
                    c3 = pltpu.async_copy(rc1_hbm.at[pl.ds(off, _B)],
                                          rc1_b, sem3)
                    c4 = pltpu.async_copy(rc2_hbm.at[pl.ds(off, _B)],
                                          rc2_b, sem1)
                    c5 = pltpu.async_copy(rc3_hbm.at[pl.ds(off, _B)],
                                          rc3_b, sem2)
                    c0.wait()
                    c1.wait()
                    c2.wait()
                    c3.wait()
                    c4.wait()
                    c5.wait()
                    lov2 = jnp.full((16,), lo, jnp.int32)
                    hiv2 = jnp.full((16,), hi, jnp.int32)
                    zrow2 = jnp.full((16,), _ZROW, jnp.int32)
                    for kk in range(_B // 16):
                        rv = recv_s[pl.ds(base + kk * 16, 16)]
                        m = (rv >= lov2) & (rv < hiv2)
                        sv = snd_b[pl.ds(kk * 16, 16)]
                        snd_b[pl.ds(kk * 16, 16)] = jnp.where(m, sv, zrow2)
                    g0 = pltpu.async_copy(ef_hbm.at[gid_b], ef_b, sem0)
                    g1 = pltpu.async_copy(h_hbm.at[snd_b], h_b, sem1)
                    g2 = pltpu.async_copy(ea_hbm.at[gid_b], ea_b, sem2)
                    g0.wait()
                    g1.wait()
                    g2.wait()

                    m_bufs = [m0_b, m1_b, m2_b, m3_b]

                    def edge_body(e, carry2):
                        hv = [h_b[e, pl.ds(c * 16, 16)]
                              for c in range(_C // 16)]
                        for l in range(_L2):
                            eav = ea_b[e, pl.ds(l * 16, 16)]
                            for c in range(_C // 16):
                                o = l * _C + c * 16
                                m_bufs[l][e, pl.ds(c * 16, 16)] = (
                                    ef_b[e, pl.ds(o, 16)] * hv[c] * eav)
                        return carry2

                    lax.fori_loop(0, _B, edge_body, 0)
                    s0 = pltpu.async_copy(m0_b, msg_sh.at[rc0_b], sem0,
                                          add=True)
                    s1 = pltpu.async_copy(m1_b, msg_sh.at[rc1_b], sem1,
                                          add=True)
                    s2 = pltpu.async_copy(m2_b, msg_sh.at[rc2_b], sem2,
                                          add=True)
                    s3 = pltpu.async_copy(m3_b, msg_sh.at[rc3_b], sem3,
                                          add=True)
                    s0.wait()
                    s1.wait()
                    s2.wait()
                    s3.wait()

                return carry

            lax.fori_loop(nstart, nend, batch_body, 0)

            plsc.subcore_barrier()
            r0 = pl.multiple_of(sid * (_RPS * _L2), 8)
            pltpu.sync_copy(
                msg_sh.at[pl.ds(r0, _RPS * _L2)],
                out_hbm.at[cid, pl.ds(p * (_RP * _L2) + r0, _RPS * _L2)])

    zeros_hbm = jnp.zeros((_RP * _L2, _C), jnp.float32)
    return k(h_pad, ea_exp, ef, rcv_a, gid_a, snd_a, rc0, rc1, rc2, rc3,
             zeros_hbm)


def _matmul_h(node_feats, W_up):
    rb = 1000

    def body(x_ref, w_ref, o_ref):
        o_ref[...] = jnp.dot(x_ref[...], w_ref[...],
                             preferred_element_type=jnp.float32)

    return pl.pallas_call(
        body,
        grid=(_N // rb,),
        in_specs=[pl.BlockSpec((rb, _C), lambda i: (i, 0)),
                  pl.BlockSpec((_C, _C), lambda i: (0, 0))],
        out_specs=pl.BlockSpec((rb, _C), lambda i: (i, 0)),
        out_shape=jax.ShapeDtypeStruct((_N, _C), jnp.float32),
    )(node_feats, W_up)


def _combine_weights(W_lin, W_skip):
    scale = 1.0 / (np.sqrt(_C) * _AVG * np.sqrt(_NELEM * _C))
    lof = [0, 1, 1, 1]

    def body(wl_ref, ws_ref, o_ref):
        for l in range(_L2):
            lw = wl_ref[lof[l]] * scale
            for e in range(_NELEM):
                o_ref[l, :, e * _C:(e + 1) * _C] = jnp.dot(
                    lw, ws_ref[e, lof[l]],
                    preferred_element_type=jnp.float32)

    return pl.pallas_call(
        body,
        out_shape=jax.ShapeDtypeStruct((_L2, _C, _NELEM * _C), jnp.float32),
    )(W_lin, W_skip)


def _final(partials, node_attrs, Wfull):
    rb = 1000

    def body(p_ref, a_ref, w_ref, o_ref):
        msg = p_ref[0] + p_ref[1]
        a = a_ref[...]
        for l in range(_L2):
            msg_l = msg[:, l * _C:(l + 1) * _C]
            y = jnp.dot(msg_l, w_ref[l], preferred_element_type=jnp.float32)
            acc = a[:, 0:1] * y[:, :_C]
            for e in range(1, _NELEM):
                acc = acc + a[:, e:e + 1] * y[:, e * _C:(e + 1) * _C]
            o_ref[:, l, :] = acc

    return pl.pallas_call(
        body,
        grid=(_N // rb,),
        in_specs=[pl.BlockSpec((_NC, rb, _ROW), lambda i: (0, i, 0)),
                  pl.BlockSpec((rb, _NELEM), lambda i: (i, 0)),
                  pl.BlockSpec((_L2, _C, _NELEM * _C), lambda i: (0, 0, 0))],
        out_specs=pl.BlockSpec((rb, _L2, _C), lambda i: (i, 0, 0)),
        out_shape=jax.ShapeDtypeStruct((_N, _L2, _C), jnp.float32),
    )(partials, node_attrs, Wfull)


def kernel(node_attrs, node_feats, edge_attrs, edge_feats, edge_index,
           W_up, W_lin, W_skip):
    h = _matmul_h(node_feats, W_up)
    h_pad = jnp.pad(h, ((0, 8), (0, 0)))
    ea_exp = jnp.pad(jnp.repeat(edge_attrs, 16, axis=1),
                     ((0, 0), (0, _C - 16 * _L2)))
    recv = edge_index[0]
    perm = jnp.argsort(recv).astype(jnp.int32)
    rcv_s = jnp.take(recv, perm)
    snd_s = jnp.take(edge_index[1], perm)
    rloc4 = (rcv_s % _RP) * _L2

    def pad_e(a, v):
        return jnp.pad(a, (0, _EPAD - _E), constant_values=v)

    gid_a = pad_e(perm, 0)
    snd_a = pad_e(snd_s, _ZROW)
    rcs = [pad_e(rloc4 + l, l) for l in range(_L2)]
    flat = _sc_messages(h_pad, ea_exp, edge_feats, rcv_s, gid_a, snd_a,
                        rcs[0], rcs[1], rcs[2], rcs[3])
    partials = flat.reshape(_NC, _NPAD, _ROW)
    Wfull = _combine_weights(W_lin, W_skip)
    return _final(partials, node_attrs, Wfull)
